# baseline (speedup 1.0000x reference)
import functools

import jax
import jax.numpy as jnp
from jax import lax
from jax.experimental import pallas as pl
from jax.experimental.pallas import tpu as pltpu
from jax.experimental.pallas import tpu_sc as plsc

_NC = 2
_NS = 16
_NW = _NC * _NS
_L = 16

_ROWS = 4096
_W = 4096
_F = 1024

_K_TC = 2048
_R_TC = 512

_SC_ROWS = _ROWS - _K_TC
_RPW = _SC_ROWS // _NW
_RPW_M = _ROWS // _NW
_CB = 8
_NCHUNK = _RPW // _CB


def _sc_body(
    x_hbm,
    m_hbm,
    out_hbm,
    mout_hbm,
    in_v0,
    in_v1,
    out_v0,
    out_v1,
    m_v,
    mo_v,
    si0,
    si1,
    so0,
    so1,
):
    wid = lax.axis_index("s") * _NC + lax.axis_index("c")
    base = wid * _RPW

    in_bufs = (in_v0, in_v1)
    out_bufs = (out_v0, out_v1)
    sin = (si0, si1)
    sout = (so0, so1)

    in_copies = [None] * _NCHUNK
    out_copies = [None] * _NCHUNK
    in_copies[0] = pltpu.async_copy(
        x_hbm.at[pl.ds(base, _CB)], in_bufs[0], sin[0]
    )

    mbase = wid * _RPW_M
    for k in range(4):
        pltpu.sync_copy(m_hbm.at[pl.ds(k * _ROWS + mbase, _RPW_M)], m_v.at[k])

    def mask_blk(j, _):
        acc = m_v[0, pl.ds(j * _L, _L)]
        for k in range(1, 4):
            acc = jnp.minimum(acc, m_v[k, pl.ds(j * _L, _L)])
        mo_v[pl.ds(j * _L, _L)] = acc
        return 0

    lax.fori_loop(0, _RPW_M // _L, mask_blk, 0, unroll=True)
    pltpu.sync_copy(mo_v, mout_hbm.at[pl.ds(mbase, _RPW_M)])

    def compute(b):
        in_b, out_b = in_bufs[b], out_bufs[b]

        def row(r, _):
            def col(j, _):
                c = j * _L
                a = in_b[r, pl.ds(c, _L)] + in_b[r, pl.ds(_F + c, _L)]
                a = a + in_b[r, pl.ds(2 * _F + c, _L)]
                a = a + in_b[r, pl.ds(3 * _F + c, _L)]
                out_b[r, pl.ds(c, _L)] = a * 0.25
                return 0

            lax.fori_loop(0, _F // _L, col, 0, unroll=8)
            return 0

        lax.fori_loop(0, _CB, row, 0)

    for ci in range(_NCHUNK):
        b = ci % 2
        nb = (ci + 1) % 2
        if ci + 1 < _NCHUNK:
            in_copies[ci + 1] = pltpu.async_copy(
                x_hbm.at[pl.ds(base + (ci + 1) * _CB, _CB)], in_bufs[nb], sin[nb]
            )
        in_copies[ci].wait()
        if ci >= 2:
            out_copies[ci - 2].wait()
        compute(b)
        out_copies[ci] = pltpu.async_copy(
            out_bufs[b], out_hbm.at[pl.ds(base + ci * _CB, _CB)], sout[b]
        )
    out_copies[_NCHUNK - 2].wait()
    out_copies[_NCHUNK - 1].wait()


_sc_call = functools.partial(
    pl.kernel,
    out_type=[
        jax.ShapeDtypeStruct((_SC_ROWS, _F), jnp.float32),
        jax.ShapeDtypeStruct((_ROWS,), jnp.int32),
    ],
    mesh=plsc.VectorSubcoreMesh(core_axis_name="c", subcore_axis_name="s"),
    scratch_types=[
        pltpu.VMEM((_CB, _W), jnp.float32),
        pltpu.VMEM((_CB, _W), jnp.float32),
        pltpu.VMEM((_CB, _F), jnp.float32),
        pltpu.VMEM((_CB, _F), jnp.float32),
        pltpu.VMEM((4, _RPW_M), jnp.int32),
        pltpu.VMEM((_RPW_M,), jnp.int32),
        pltpu.SemaphoreType.DMA,
        pltpu.SemaphoreType.DMA,
        pltpu.SemaphoreType.DMA,
        pltpu.SemaphoreType.DMA,
    ],
)(_sc_body)


def _tc_body(x_ref, o_ref):
    x = x_ref[:]
    acc = x[:, 0:_F] + x[:, _F : 2 * _F]
    acc = acc + x[:, 2 * _F : 3 * _F] + x[:, 3 * _F : 4 * _F]
    o_ref[:] = acc * 0.25


def kernel(dense_x, dense_padding_mask):
    bsz, tsz, fsz = dense_x.shape
    gs = 4
    tn = tsz // gs

    x2 = dense_x.reshape(_ROWS, _W)
    m4 = (
        dense_padding_mask.reshape(_ROWS, gs)
        .astype(jnp.int32)
        .T.reshape(gs * _ROWS)
    )

    out_tc = pl.pallas_call(
        _tc_body,
        grid=(_K_TC // _R_TC,),
        in_specs=[pl.BlockSpec((_R_TC, _W), lambda i: (i, 0))],
        out_specs=pl.BlockSpec((_R_TC, _F), lambda i: (i, 0)),
        out_shape=jax.ShapeDtypeStruct((_ROWS, _F), jnp.float32),
    )(x2[:_K_TC])

    out_sc, mout = _sc_call(x2[_K_TC:], m4)

    out = lax.dynamic_update_slice(out_tc, out_sc, (_K_TC, 0))
    return (
        out.reshape(bsz, tn, fsz),
        mout.reshape(bsz, tn).astype(jnp.bool_),
    )


# speedup vs baseline: 1.6049x; 1.6049x over previous
import functools

import jax
import jax.numpy as jnp
from jax import lax
from jax.experimental import pallas as pl
from jax.experimental.pallas import tpu as pltpu
from jax.experimental.pallas import tpu_sc as plsc

_NC = 2
_NS = 16
_NW = _NC * _NS
_L = 16

_ROWS = 4096
_W = 4096
_F = 1024

_K_TC = 2048
_R_TC = 512

_SC_ROWS = _ROWS - _K_TC
_RPW = _SC_ROWS // _NW
_RPW_M = _ROWS // _NW
_CB = 8
_NCHUNK = _RPW // _CB


def _sc_body(
    x_hbm,
    m_hbm,
    out_hbm,
    mout_hbm,
    in_v0,
    in_v1,
    out_v0,
    out_v1,
    m_v,
    mo_v,
    si0,
    si1,
    so0,
    so1,
):
    wid = lax.axis_index("s") * _NC + lax.axis_index("c")
    base = _K_TC + wid * _RPW
    obase = wid * _RPW

    in_bufs = (in_v0, in_v1)
    out_bufs = (out_v0, out_v1)
    sin = (si0, si1)
    sout = (so0, so1)

    in_copies = [None] * _NCHUNK
    out_copies = [None] * _NCHUNK
    in_copies[0] = pltpu.async_copy(
        x_hbm.at[pl.ds(base, _CB)], in_bufs[0], sin[0]
    )

    mbase = wid * _RPW_M
    for k in range(4):
        pltpu.sync_copy(m_hbm.at[pl.ds(k * _ROWS + mbase, _RPW_M)], m_v.at[k])

    def mask_blk(j, _):
        acc = m_v[0, pl.ds(j * _L, _L)]
        for k in range(1, 4):
            acc = jnp.minimum(acc, m_v[k, pl.ds(j * _L, _L)])
        mo_v[pl.ds(j * _L, _L)] = acc
        return 0

    lax.fori_loop(0, _RPW_M // _L, mask_blk, 0, unroll=True)
    pltpu.sync_copy(mo_v, mout_hbm.at[pl.ds(mbase, _RPW_M)])

    def compute(b):
        in_b, out_b = in_bufs[b], out_bufs[b]

        def row(r, _):
            def col(j, _):
                c = j * _L
                a = in_b[r, pl.ds(c, _L)] + in_b[r, pl.ds(_F + c, _L)]
                a = a + in_b[r, pl.ds(2 * _F + c, _L)]
                a = a + in_b[r, pl.ds(3 * _F + c, _L)]
                out_b[r, pl.ds(c, _L)] = a * 0.25
                return 0

            lax.fori_loop(0, _F // _L, col, 0, unroll=8)
            return 0

        lax.fori_loop(0, _CB, row, 0)

    for ci in range(_NCHUNK):
        b = ci % 2
        nb = (ci + 1) % 2
        if ci + 1 < _NCHUNK:
            in_copies[ci + 1] = pltpu.async_copy(
                x_hbm.at[pl.ds(base + (ci + 1) * _CB, _CB)], in_bufs[nb], sin[nb]
            )
        in_copies[ci].wait()
        if ci >= 2:
            out_copies[ci - 2].wait()
        compute(b)
        out_copies[ci] = pltpu.async_copy(
            out_bufs[b], out_hbm.at[pl.ds(obase + ci * _CB, _CB)], sout[b]
        )
    out_copies[_NCHUNK - 2].wait()
    out_copies[_NCHUNK - 1].wait()


_sc_call = functools.partial(
    pl.kernel,
    out_type=[
        jax.ShapeDtypeStruct((_SC_ROWS, _F), jnp.float32),
        jax.ShapeDtypeStruct((_ROWS,), jnp.int32),
    ],
    mesh=plsc.VectorSubcoreMesh(core_axis_name="c", subcore_axis_name="s"),
    scratch_types=[
        pltpu.VMEM((_CB, _W), jnp.float32),
        pltpu.VMEM((_CB, _W), jnp.float32),
        pltpu.VMEM((_CB, _F), jnp.float32),
        pltpu.VMEM((_CB, _F), jnp.float32),
        pltpu.VMEM((4, _RPW_M), jnp.int32),
        pltpu.VMEM((_RPW_M,), jnp.int32),
        pltpu.SemaphoreType.DMA,
        pltpu.SemaphoreType.DMA,
        pltpu.SemaphoreType.DMA,
        pltpu.SemaphoreType.DMA,
    ],
)(_sc_body)


def _tc_body(x_ref, o_ref):
    x = x_ref[:]
    acc = x[:, 0:_F] + x[:, _F : 2 * _F]
    acc = acc + x[:, 2 * _F : 3 * _F] + x[:, 3 * _F : 4 * _F]
    o_ref[:] = acc * 0.25


def kernel(dense_x, dense_padding_mask):
    bsz, tsz, fsz = dense_x.shape
    gs = 4
    tn = tsz // gs

    x2 = dense_x.reshape(_ROWS, _W)
    m4 = (
        dense_padding_mask.reshape(_ROWS, gs)
        .astype(jnp.int32)
        .T.reshape(gs * _ROWS)
    )

    out_tc = pl.pallas_call(
        _tc_body,
        grid=(_K_TC // _R_TC,),
        in_specs=[pl.BlockSpec((_R_TC, _W), lambda i: (i, 0))],
        out_specs=pl.BlockSpec((_R_TC, _F), lambda i: (i, 0)),
        out_shape=jax.ShapeDtypeStruct((_ROWS, _F), jnp.float32),
    )(x2)

    out_sc, mout = _sc_call(x2, m4)

    out = lax.dynamic_update_slice(out_tc, out_sc, (_K_TC, 0))
    return (
        out.reshape(bsz, tn, fsz),
        mout.reshape(bsz, tn).astype(jnp.bool_),
    )


# speedup vs baseline: 1.6191x; 1.0088x over previous
import functools

import jax
import jax.numpy as jnp
from jax import lax
from jax.experimental import pallas as pl
from jax.experimental.pallas import tpu as pltpu
from jax.experimental.pallas import tpu_sc as plsc

_NC = 2
_NS = 16
_NW = _NC * _NS
_L = 16

_ROWS = 4096
_W = 4096
_F = 1024

_K_TC = 2048
_R_TC = 512

_SC_ROWS = _ROWS - _K_TC
_RPW = _SC_ROWS // _NW
_RPW_M = _ROWS // _NW
_CB = 8
_NCHUNK = _RPW // _CB


def _sc_body(
    x_hbm,
    m_hbm,
    out_hbm,
    mout_hbm,
    in_v0,
    in_v1,
    out_v0,
    out_v1,
    m_v,
    mo_v,
    si0,
    si1,
    so0,
    so1,
):
    wid = lax.axis_index("s") * _NC + lax.axis_index("c")
    base = _K_TC + wid * _RPW
    obase = wid * _RPW

    in_bufs = (in_v0, in_v1)
    out_bufs = (out_v0, out_v1)
    sin = (si0, si1)
    sout = (so0, so1)

    in_copies = [None] * _NCHUNK
    out_copies = [None] * _NCHUNK
    in_copies[0] = pltpu.async_copy(
        x_hbm.at[pl.ds(base, _CB)], in_bufs[0], sin[0]
    )

    mbase = wid * _RPW_M
    for k in range(4):
        pltpu.sync_copy(m_hbm.at[pl.ds(k * _ROWS + mbase, _RPW_M)], m_v.at[k])

    def mask_blk(j, _):
        acc = m_v[0, pl.ds(j * _L, _L)]
        for k in range(1, 4):
            acc = jnp.minimum(acc, m_v[k, pl.ds(j * _L, _L)])
        mo_v[pl.ds(j * _L, _L)] = acc
        return 0

    lax.fori_loop(0, _RPW_M // _L, mask_blk, 0, unroll=True)
    pltpu.sync_copy(mo_v, mout_hbm.at[pl.ds(mbase, _RPW_M)])

    def compute(b):
        in_b, out_b = in_bufs[b], out_bufs[b]

        def row(r, _):
            def col(j, _):
                c = j * _L
                a = in_b[r, pl.ds(c, _L)] + in_b[r, pl.ds(_F + c, _L)]
                a = a + in_b[r, pl.ds(2 * _F + c, _L)]
                a = a + in_b[r, pl.ds(3 * _F + c, _L)]
                out_b[r, pl.ds(c, _L)] = a * 0.25
                return 0

            lax.fori_loop(0, _F // _L, col, 0, unroll=8)
            return 0

        lax.fori_loop(0, _CB, row, 0)

    for ci in range(_NCHUNK):
        b = ci % 2
        nb = (ci + 1) % 2
        if ci + 1 < _NCHUNK:
            in_copies[ci + 1] = pltpu.async_copy(
                x_hbm.at[pl.ds(base + (ci + 1) * _CB, _CB)], in_bufs[nb], sin[nb]
            )
        in_copies[ci].wait()
        if ci >= 2:
            out_copies[ci - 2].wait()
        compute(b)
        out_copies[ci] = pltpu.async_copy(
            out_bufs[b], out_hbm.at[pl.ds(obase + ci * _CB, _CB)], sout[b]
        )
    out_copies[_NCHUNK - 2].wait()
    out_copies[_NCHUNK - 1].wait()


_sc_call = functools.partial(
    pl.kernel,
    out_type=[
        jax.ShapeDtypeStruct((_SC_ROWS, _F), jnp.float32),
        jax.ShapeDtypeStruct((_ROWS,), jnp.int32),
    ],
    mesh=plsc.VectorSubcoreMesh(core_axis_name="c", subcore_axis_name="s"),
    scratch_types=[
        pltpu.VMEM((_CB, _W), jnp.float32),
        pltpu.VMEM((_CB, _W), jnp.float32),
        pltpu.VMEM((_CB, _F), jnp.float32),
        pltpu.VMEM((_CB, _F), jnp.float32),
        pltpu.VMEM((4, _RPW_M), jnp.int32),
        pltpu.VMEM((_RPW_M,), jnp.int32),
        pltpu.SemaphoreType.DMA,
        pltpu.SemaphoreType.DMA,
        pltpu.SemaphoreType.DMA,
        pltpu.SemaphoreType.DMA,
    ],
)(_sc_body)


def _stitch_body(dst_ref, src_ref, out_ref):
    del dst_ref
    out_ref[:] = src_ref[:]


def _tc_body(x_ref, o_ref):
    x = x_ref[:]
    acc = x[:, 0:_F] + x[:, _F : 2 * _F]
    acc = acc + x[:, 2 * _F : 3 * _F] + x[:, 3 * _F : 4 * _F]
    o_ref[:] = acc * 0.25


def kernel(dense_x, dense_padding_mask):
    bsz, tsz, fsz = dense_x.shape
    gs = 4
    tn = tsz // gs

    x2 = dense_x.reshape(_ROWS, _W)
    m4 = (
        dense_padding_mask.reshape(_ROWS, gs)
        .astype(jnp.int32)
        .T.reshape(gs * _ROWS)
    )

    out_tc = pl.pallas_call(
        _tc_body,
        grid=(_K_TC // _R_TC,),
        in_specs=[pl.BlockSpec((_R_TC, _W), lambda i: (i, 0))],
        out_specs=pl.BlockSpec((_R_TC, _F), lambda i: (i, 0)),
        out_shape=jax.ShapeDtypeStruct((_ROWS, _F), jnp.float32),
    )(x2)

    out_sc, mout = _sc_call(x2, m4)

    out = pl.pallas_call(
        _stitch_body,
        grid=(_SC_ROWS // _R_TC,),
        in_specs=[
            pl.BlockSpec((8, 128), lambda i: (0, 0)),
            pl.BlockSpec((_R_TC, _F), lambda i: (i, 0)),
        ],
        out_specs=pl.BlockSpec((_R_TC, _F), lambda i: (i + _K_TC // _R_TC, 0)),
        out_shape=jax.ShapeDtypeStruct((_ROWS, _F), jnp.float32),
        input_output_aliases={0: 0},
    )(out_tc, out_sc)
    return (
        out.reshape(bsz, tn, fsz),
        mout.reshape(bsz, tn).astype(jnp.bool_),
    )


# speedup vs baseline: 3.1251x; 1.9301x over previous
import functools

import jax
import jax.numpy as jnp
from jax import lax
from jax.experimental import pallas as pl
from jax.experimental.pallas import tpu as pltpu
from jax.experimental.pallas import tpu_sc as plsc

_NC = 2
_NS = 16
_NW = _NC * _NS
_L = 16

_B = 4
_T = 4096
_F = 1024
_GS = 4
_TN = _T // _GS
_WINDOWS = _B * _TN
_RPW_M = _WINDOWS // _NW

_S = 2048


def _sc_mask_body(m_hbm, mout_hbm, m_v, mo_v):
    wid = lax.axis_index("s") * _NC + lax.axis_index("c")
    base = wid * _RPW_M

    for k in range(_GS):
        pltpu.sync_copy(m_hbm.at[pl.ds(k * _WINDOWS + base, _RPW_M)], m_v.at[k])

    def mask_blk(j, _):
        acc = m_v[0, pl.ds(j * _L, _L)]
        for k in range(1, _GS):
            acc = jnp.minimum(acc, m_v[k, pl.ds(j * _L, _L)])
        mo_v[pl.ds(j * _L, _L)] = acc
        return 0

    lax.fori_loop(0, _RPW_M // _L, mask_blk, 0, unroll=True)
    pltpu.sync_copy(mo_v, mout_hbm.at[pl.ds(base, _RPW_M)])


_sc_mask = functools.partial(
    pl.kernel,
    out_type=jax.ShapeDtypeStruct((_WINDOWS,), jnp.int32),
    mesh=plsc.VectorSubcoreMesh(core_axis_name="c", subcore_axis_name="s"),
    scratch_types=[
        pltpu.VMEM((_GS, _RPW_M), jnp.int32),
        pltpu.VMEM((_RPW_M,), jnp.int32),
    ],
)(_sc_mask_body)


def _tc_body(x_ref, o_ref):
    x = x_ref[0].reshape(_S // _GS, _GS, _F)
    o_ref[0] = jnp.sum(x, axis=1) * (1.0 / _GS)


def kernel(dense_x, dense_padding_mask):
    bsz, tsz, fsz = dense_x.shape

    m4 = (
        dense_padding_mask.reshape(_WINDOWS, _GS)
        .astype(jnp.int32)
        .T.reshape(_GS * _WINDOWS)
    )
    mout = _sc_mask(m4)

    out = pl.pallas_call(
        _tc_body,
        grid=(_B, _T // _S),
        in_specs=[pl.BlockSpec((1, _S, _F), lambda b, j: (b, j, 0))],
        out_specs=pl.BlockSpec((1, _S // _GS, _F), lambda b, j: (b, j, 0)),
        out_shape=jax.ShapeDtypeStruct((_B, _TN, _F), jnp.float32),
    )(dense_x)

    return (out, mout.reshape(bsz, _TN).astype(jnp.bool_))


# speedup vs baseline: 4.1017x; 1.3125x over previous
import functools

import jax
import jax.numpy as jnp
from jax import lax
from jax.experimental import pallas as pl
from jax.experimental.pallas import tpu as pltpu
from jax.experimental.pallas import tpu_sc as plsc

_NC = 2
_NS = 16
_NW = _NC * _NS
_L = 16

_B = 4
_T = 4096
_F = 1024
_GS = 4
_TN = _T // _GS
_WINDOWS = _B * _TN
_RPW_M = _WINDOWS // _NW

_S = 2048


def _sc_mask_body(m_hbm, mout_hbm, m_v, mo_v):
    wid = lax.axis_index("s") * _NC + lax.axis_index("c")
    base = wid * _RPW_M

    for k in range(_GS):
        pltpu.sync_copy(m_hbm.at[pl.ds(k * _WINDOWS + base, _RPW_M)], m_v.at[k])

    def mask_blk(j, _):
        acc = m_v[0, pl.ds(j * _L, _L)]
        for k in range(1, _GS):
            acc = jnp.minimum(acc, m_v[k, pl.ds(j * _L, _L)])
        mo_v[pl.ds(j * _L, _L)] = acc
        return 0

    lax.fori_loop(0, _RPW_M // _L, mask_blk, 0, unroll=True)
    pltpu.sync_copy(mo_v, mout_hbm.at[pl.ds(base, _RPW_M)])


_sc_mask = functools.partial(
    pl.kernel,
    out_type=jax.ShapeDtypeStruct((_WINDOWS,), jnp.int32),
    mesh=plsc.VectorSubcoreMesh(core_axis_name="c", subcore_axis_name="s"),
    scratch_types=[
        pltpu.VMEM((_GS, _RPW_M), jnp.int32),
        pltpu.VMEM((_RPW_M,), jnp.int32),
    ],
)(_sc_mask_body)


def _tc_body(x_ref, o_ref):
    x = x_ref[0].reshape(_S // 2, 2, _F)
    r1 = x[:, 0, :] + x[:, 1, :]
    r2 = r1.reshape(_S // 4, 2, _F)
    o_ref[0] = (r2[:, 0, :] + r2[:, 1, :]) * (1.0 / _GS)


def kernel(dense_x, dense_padding_mask):
    bsz, tsz, fsz = dense_x.shape

    m4 = (
        dense_padding_mask.reshape(_WINDOWS, _GS)
        .astype(jnp.int32)
        .T.reshape(_GS * _WINDOWS)
    )
    mout = _sc_mask(m4)

    out = pl.pallas_call(
        _tc_body,
        grid=(_B, _T // _S),
        in_specs=[pl.BlockSpec((1, _S, _F), lambda b, j: (b, j, 0))],
        out_specs=pl.BlockSpec((1, _S // _GS, _F), lambda b, j: (b, j, 0)),
        out_shape=jax.ShapeDtypeStruct((_B, _TN, _F), jnp.float32),
    )(dense_x)

    return (out, mout.reshape(bsz, _TN).astype(jnp.bool_))


# speedup vs baseline: 4.1392x; 1.0091x over previous
import functools

import jax
import jax.numpy as jnp
from jax import lax
from jax.experimental import pallas as pl
from jax.experimental.pallas import tpu as pltpu
from jax.experimental.pallas import tpu_sc as plsc

_NC = 2
_NS = 16
_NW = _NC * _NS
_L = 16

_B = 4
_T = 4096
_F = 1024
_GS = 4
_TN = _T // _GS
_WINDOWS = _B * _TN
_RPW_M = _WINDOWS // _NW

_S = 4096


def _sc_mask_body(m_hbm, mout_hbm, m_v, mo_v):
    wid = lax.axis_index("s") * _NC + lax.axis_index("c")
    base = wid * _RPW_M

    for k in range(_GS):
        pltpu.sync_copy(m_hbm.at[pl.ds(k * _WINDOWS + base, _RPW_M)], m_v.at[k])

    def mask_blk(j, _):
        acc = m_v[0, pl.ds(j * _L, _L)]
        for k in range(1, _GS):
            acc = jnp.minimum(acc, m_v[k, pl.ds(j * _L, _L)])
        mo_v[pl.ds(j * _L, _L)] = acc
        return 0

    lax.fori_loop(0, _RPW_M // _L, mask_blk, 0, unroll=True)
    pltpu.sync_copy(mo_v, mout_hbm.at[pl.ds(base, _RPW_M)])


_sc_mask = functools.partial(
    pl.kernel,
    out_type=jax.ShapeDtypeStruct((_WINDOWS,), jnp.int32),
    mesh=plsc.VectorSubcoreMesh(core_axis_name="c", subcore_axis_name="s"),
    scratch_types=[
        pltpu.VMEM((_GS, _RPW_M), jnp.int32),
        pltpu.VMEM((_RPW_M,), jnp.int32),
    ],
)(_sc_mask_body)


def _tc_body(x_ref, o_ref):
    x = x_ref[0].reshape(_S // 2, 2, _F)
    r1 = x[:, 0, :] + x[:, 1, :]
    r2 = r1.reshape(_S // 4, 2, _F)
    o_ref[0] = (r2[:, 0, :] + r2[:, 1, :]) * (1.0 / _GS)


def kernel(dense_x, dense_padding_mask):
    bsz, tsz, fsz = dense_x.shape

    m4 = (
        dense_padding_mask.reshape(_WINDOWS, _GS)
        .astype(jnp.int32)
        .T.reshape(_GS * _WINDOWS)
    )
    mout = _sc_mask(m4)

    out = pl.pallas_call(
        _tc_body,
        grid=(_B, _T // _S),
        in_specs=[pl.BlockSpec((1, _S, _F), lambda b, j: (b, j, 0))],
        out_specs=pl.BlockSpec((1, _S // _GS, _F), lambda b, j: (b, j, 0)),
        out_shape=jax.ShapeDtypeStruct((_B, _TN, _F), jnp.float32),
    )(dense_x)

    return (out, mout.reshape(bsz, _TN).astype(jnp.bool_))
